# Initial kernel scaffold; baseline (speedup 1.0000x reference)
import jax
import jax.numpy as jnp
from jax.experimental import pallas as pl


def kernel(logits, label):
    raise NotImplementedError("write your pallas kernel here")


# speedup vs baseline: 5.6745x; 5.6745x over previous
import jax
import jax.numpy as jnp
from jax.experimental import pallas as pl

_S = 64.0
_M = 0.35
_STD = 0.0125
_BR = 16


def _body(lab_ref, marg_ref, x_ref, o_ref):
    x = x_ref[...]
    cols = jax.lax.broadcasted_iota(jnp.int32, x.shape, 1)
    corr = jnp.where(cols == lab_ref[...], marg_ref[...], 0.0)
    o_ref[...] = x * _S - corr


def kernel(logits, label):
    n, c = logits.shape
    lab = label.astype(jnp.int32).reshape(n, 1)
    marg = (jax.random.normal(jax.random.key(42), (n, 1), jnp.float32) * _STD + _M) * _S
    return pl.pallas_call(
        _body,
        grid=(n // _BR,),
        in_specs=[
            pl.BlockSpec((_BR, 1), lambda i: (i, 0)),
            pl.BlockSpec((_BR, 1), lambda i: (i, 0)),
            pl.BlockSpec((_BR, c), lambda i: (i, 0)),
        ],
        out_specs=pl.BlockSpec((_BR, c), lambda i: (i, 0)),
        out_shape=jax.ShapeDtypeStruct((n, c), logits.dtype),
    )(lab, marg, logits)
